# baseline (speedup 1.0000x reference)
import functools

import jax
import jax.numpy as jnp
from jax import lax
from jax.experimental import pallas as pl
from jax.experimental.pallas import tpu as pltpu
from jax.experimental.pallas import tpu_sc as plsc

EMB = 128
TOTAL_ROWS = 4096 * 200
CH = 128
PER = 8


def _make_sc_lookup(total_rows, emb):
    info = plsc.get_sparse_core_info()
    nc, ns = info.num_cores, info.num_subcores
    nw = nc * ns
    rows_per_w = total_rows // nw
    n_steps = rows_per_w // CH
    n_periods = n_steps // PER

    mesh = plsc.VectorSubcoreMesh(core_axis_name="c", subcore_axis_name="s")

    @functools.partial(
        pl.kernel,
        mesh=mesh,
        out_type=jax.ShapeDtypeStruct((total_rows, emb), jnp.float32),
        scratch_types=[
            pltpu.VMEM_SHARED((8, emb), jnp.float32),
            pltpu.VMEM((8, emb), jnp.float32),
            pltpu.VMEM((n_steps, CH), jnp.int32),
            pltpu.VMEM((4, CH, emb), jnp.float32),
            pltpu.VMEM((2, CH, emb), jnp.float32),
            pltpu.SemaphoreType.DMA,
            pltpu.SemaphoreType.DMA,
            pltpu.SemaphoreType.DMA,
            pltpu.SemaphoreType.DMA,
            pltpu.SemaphoreType.DMA,
            pltpu.SemaphoreType.DMA,
            pltpu.SemaphoreType.DMA,
            pltpu.SemaphoreType.DMA,
        ],
    )
    def k(idx_hbm, table_hbm, out_hbm, table_sh, table_v, idx_v, sb, vb,
          gsem, isem, vw0, vw1, sw0, sw1, sw2, sw3):
        vws = (vw0, vw1)
        sws = (sw0, sw1, sw2, sw3)
        c = lax.axis_index("c")
        s = lax.axis_index("s")
        wid = s * nc + c
        base = wid * rows_per_w

        @pl.when(s == 0)
        def _():
            pltpu.sync_copy(table_hbm, table_sh.at[pl.ds(0, 3)])

        pltpu.sync_copy(table_hbm, table_v.at[pl.ds(0, 3)])
        pltpu.sync_copy(idx_hbm.at[wid], idx_v)
        plsc.subcore_barrier()

        def gather(step, k_buf):
            pltpu.async_copy(table_sh.at[idx_v.at[step]], sb.at[k_buf], gsem)

        def wait_gather(k_buf):
            pltpu.make_async_copy(
                table_sh.at[idx_v.at[0]], sb.at[k_buf], gsem
            ).wait()

        def write_s(step, k_buf):
            pltpu.async_copy(
                sb.at[k_buf], out_hbm.at[pl.ds(base + step * CH, CH)], sws[k_buf]
            )

        def wait_write_s(k_buf):
            pltpu.make_async_copy(
                sb.at[k_buf], out_hbm.at[pl.ds(base, CH)], sws[k_buf]
            ).wait()

        def write_v(step, slot):
            pltpu.async_copy(
                vb.at[slot], out_hbm.at[pl.ds(base + step * CH, CH)], vws[slot]
            )

        def wait_write_v(slot):
            pltpu.make_async_copy(
                vb.at[slot], out_hbm.at[pl.ds(base, CH)], vws[slot]
            ).wait()

        def expand(row, slot):
            def rb(i, car):
                grp = idx_v[row, pl.ds(i * 16, 16)]
                for u in range(16):
                    sc = grp[u]
                    rr = i * 16 + u
                    for j in range(8):
                        vb[slot, rr, pl.ds(j * 16, 16)] = (
                            table_v[sc, pl.ds(j * 16, 16)]
                        )
                return car
            lax.fori_loop(0, CH // 16, rb, 0)

        def period_body(pp, car):
            c0 = pp * PER
            for kb in range(4):
                @pl.when(pp > 0)
                def _():
                    wait_write_s(kb)

                gather(c0 + kb, kb)

            @pl.when(pp > 0)
            def _():
                wait_write_v(0)

            expand(c0 + 6, 0)
            write_v(c0 + 6, 0)
            wait_gather(0)
            write_s(c0 + 0, 0)
            wait_gather(1)
            write_s(c0 + 1, 1)

            @pl.when(pp > 0)
            def _():
                wait_write_v(1)

            expand(c0 + 7, 1)
            write_v(c0 + 7, 1)
            wait_gather(2)
            write_s(c0 + 2, 2)
            wait_gather(3)
            write_s(c0 + 3, 3)
            wait_write_s(0)
            gather(c0 + 4, 0)
            wait_write_s(1)
            gather(c0 + 5, 1)
            wait_gather(0)
            write_s(c0 + 4, 0)
            wait_gather(1)
            write_s(c0 + 5, 1)
            return car

        lax.fori_loop(0, n_periods, period_body, 0)
        for kb in range(4):
            wait_write_s(kb)
        for slot in range(2):
            wait_write_v(slot)

    return k, nc, nw, rows_per_w, n_steps


_sc_lookup, _NC, _NW, _RPW, _NSTEPS = _make_sc_lookup(TOTAL_ROWS, EMB)


def kernel(indices, table):
    B, T = indices.shape
    flat = indices.reshape(B * T).astype(jnp.int32)
    out = _sc_lookup(flat.reshape(_NW, _NSTEPS, CH), table)
    return out.reshape(B, T, EMB)


# speedup vs baseline: 1.4725x; 1.4725x over previous
import functools

import jax
import jax.numpy as jnp
from jax import lax
from jax.experimental import pallas as pl
from jax.experimental.pallas import tpu as pltpu
from jax.experimental.pallas import tpu_sc as plsc

EMB = 128
TOTAL_ROWS = 4096 * 200
CH = 128
NB = 4
GA = 2


def _make_sc_lookup(total_rows, emb):
    info = plsc.get_sparse_core_info()
    nc, ns = info.num_cores, info.num_subcores
    nw = nc * ns
    rows_per_w = total_rows // nw
    n_steps = rows_per_w // CH
    n_groups = n_steps // NB

    mesh = plsc.VectorSubcoreMesh(core_axis_name="c", subcore_axis_name="s")

    @functools.partial(
        pl.kernel,
        mesh=mesh,
        out_type=jax.ShapeDtypeStruct((total_rows, emb), jnp.float32),
        scratch_types=[
            pltpu.VMEM_SHARED((8, emb), jnp.float32),
            pltpu.VMEM((n_steps, CH), jnp.int32),
            pltpu.VMEM((NB, CH, emb), jnp.float32),
            pltpu.SemaphoreType.DMA,
        ] + [pltpu.SemaphoreType.DMA] * NB,
    )
    def k(idx_hbm, table_hbm, out_hbm, table_sh, idx_v, rows_v, gsem, *wsems):
        c = lax.axis_index("c")
        s = lax.axis_index("s")
        wid = s * nc + c
        base = wid * rows_per_w
        @pl.when(s == 0)
        def _():
            pltpu.sync_copy(table_hbm, table_sh.at[pl.ds(0, 3)])

        pltpu.sync_copy(idx_hbm.at[wid], idx_v)
        plsc.subcore_barrier()

        def gather(step, buf):
            pltpu.async_copy(table_sh.at[idx_v.at[step]], rows_v.at[buf], gsem)

        def wait_gather(buf):
            pltpu.make_async_copy(
                table_sh.at[idx_v.at[0]], rows_v.at[buf], gsem
            ).wait()

        def wait_write(buf):
            pltpu.make_async_copy(
                rows_v.at[buf], out_hbm.at[pl.ds(base, CH)], wsems[buf]
            ).wait()

        for p in range(GA):
            gather(p, p)

        def group(g, carry):
            for b in range(NB):
                st = g * NB + b

                nb = (b + GA) % NB

                @pl.when(st + GA < n_steps)
                def _():

                    @pl.when(st + GA >= NB)
                    def _():
                        wait_write(nb)

                    gather(st + GA, nb)

                wait_gather(b)
                pltpu.async_copy(
                    rows_v.at[b], out_hbm.at[pl.ds(base + st * CH, CH)], wsems[b]
                )
            return carry

        lax.fori_loop(0, n_groups, group, 0)
        for b in range(NB):
            wait_write(b)

    return k, nc, nw, rows_per_w, n_steps


_sc_lookup, _NC, _NW, _RPW, _NSTEPS = _make_sc_lookup(TOTAL_ROWS, EMB)


def kernel(indices, table):
    B, T = indices.shape
    flat = indices.reshape(B * T).astype(jnp.int32)
    out = _sc_lookup(flat.reshape(_NW, _NSTEPS, CH), table)
    return out.reshape(B, T, EMB)
